# baseline (speedup 1.0000x reference)
import functools

import jax
import jax.numpy as jnp
from jax import lax
from jax.experimental import pallas as pl
from jax.experimental.pallas import tpu as pltpu
from jax.experimental.pallas import tpu_sc as plsc

N = 16384
D = 64
N_REL = 10
NC = 2
NS = 16
NW = NC * NS
TPW = N // NW
IC = 128
NCHUNK = TPW // IC
LANES = 16
CH = D // LANES
GPC = IC // LANES
NG = TPW // LANES
UNROLL = 8


def _body(rel_idx, ent_idx, val_idx, rel_tab, ent_tab, out,
          ridx_v, eidx_v, vidx_v, rtab_v, erows_v, vrows_v, acc_v, part_v,
          sem0, sem1, sem2, sem3):
    sems = (sem0, sem1, sem2, sem3)
    wid = lax.axis_index("s") * NC + lax.axis_index("c")
    base = wid * TPW

    pltpu.sync_copy(rel_idx.at[pl.ds(base, TPW)], ridx_v.at[pl.ds(0, TPW)])
    pltpu.sync_copy(ent_idx.at[pl.ds(base, TPW)], eidx_v)
    pltpu.sync_copy(val_idx.at[pl.ds(base, TPW)], vidx_v)
    pltpu.sync_copy(rel_tab, rtab_v)

    copies = []
    for j in range(NCHUNK):
        sl = pl.ds(j * IC, IC)
        copies.append((
            pltpu.async_copy(ent_tab.at[eidx_v.at[sl]], erows_v.at[sl], sems[j]),
            pltpu.async_copy(ent_tab.at[vidx_v.at[sl]], vrows_v.at[sl], sems[j]),
        ))

    z = jnp.zeros((LANES,), jnp.float32)

    def zbody(r, _):
        for c in range(CH):
            acc_v[r, pl.ds(c * LANES, LANES)] = z
        return 0

    lax.fori_loop(0, N_REL, zbody, 0)

    copies[0][0].wait()
    copies[0][1].wait()

    def gbody(g, _):
        for j in range(1, NCHUNK):
            @pl.when(g == j * (IC // UNROLL))
            def _wait(j=j):
                copies[j][0].wait()
                copies[j][1].wait()

        rv = ridx_v[pl.ds(g * UNROLL, LANES)]
        for l in range(UNROLL):
            r = rv[l]
            i = g * UNROLL + l
            for c in range(CH):
                sl = pl.ds(c * LANES, LANES)
                plsc.addupdate(acc_v.at[r, sl], erows_v[i, sl] + vrows_v[i, sl])
        return 0

    lax.fori_loop(0, TPW // UNROLL, gbody, 0)

    def fbody(r, part):
        out_part = []
        for c in range(CH):
            sl = pl.ds(c * LANES, LANES)
            out_part.append(part[c] + rtab_v[r, sl] * acc_v[r, sl])
        return tuple(out_part)

    part = lax.fori_loop(0, N_REL, fbody, (z,) * CH)
    for c in range(CH):
        part_v[pl.ds(c * LANES, LANES)] = part[c]
    pltpu.sync_copy(part_v, out.at[wid])


@jax.jit
def kernel(rel_idx, ent_idx, val_idx, relation_embed, entity_embed):
    mesh = plsc.VectorSubcoreMesh(core_axis_name="c", subcore_axis_name="s")
    k = functools.partial(
        pl.kernel,
        mesh=mesh,
        compiler_params=pltpu.CompilerParams(use_tc_tiling_on_sc=False),
        out_type=jax.ShapeDtypeStruct((NW, D), jnp.float32),
        scratch_types=[
            pltpu.VMEM((TPW + LANES,), jnp.int32),
            pltpu.VMEM((TPW,), jnp.int32),
            pltpu.VMEM((TPW,), jnp.int32),
            pltpu.VMEM((N_REL, D), jnp.float32),
            pltpu.VMEM((TPW, D), jnp.float32),
            pltpu.VMEM((TPW, D), jnp.float32),
            pltpu.VMEM((N_REL, D), jnp.float32),
            pltpu.VMEM((D,), jnp.float32),
            pltpu.SemaphoreType.DMA,
            pltpu.SemaphoreType.DMA,
            pltpu.SemaphoreType.DMA,
            pltpu.SemaphoreType.DMA,
        ],
    )(_body)
    partials = k(rel_idx, ent_idx, val_idx, relation_embed, entity_embed)
    return partials.sum(axis=0) * (1.0 / N)


# speedup vs baseline: 1.0729x; 1.0729x over previous
import functools

import jax
import jax.numpy as jnp
import numpy as np
from jax import lax
from jax.experimental import pallas as pl
from jax.experimental.pallas import tpu as pltpu
from jax.experimental.pallas import tpu_sc as plsc

N = 16384
D = 64
N_REL = 10
NC = 2
NS = 16
NW = NC * NS
TPW = N // NW
IC = 128
NCHUNK = TPW // IC
LANES = 16
CH = D // LANES
GPC = IC // LANES
NG = TPW // LANES

_PERM = np.concatenate(
    [np.concatenate([np.arange(0, 32, 2), np.arange(1, 32, 2)]) + 32 * c
     for c in range(D // 32)]
)
_INV_PERM = np.argsort(_PERM)


def _body(rel_idx, ent_idx, val_idx, rel_tab, ent_tab, out,
          ridx_v, eidx_v, vidx_v, rtab_v, erows_v, vrows_v, acc_v, part_v,
          sem0, sem1, sem2, sem3):
    sems = (sem0, sem1, sem2, sem3)
    wid = lax.axis_index("s") * NC + lax.axis_index("c")
    base = wid * TPW

    pltpu.sync_copy(rel_idx.at[pl.ds(base, TPW)], ridx_v)
    pltpu.sync_copy(ent_idx.at[pl.ds(base, TPW)], eidx_v)
    pltpu.sync_copy(val_idx.at[pl.ds(base, TPW)], vidx_v)
    pltpu.sync_copy(rel_tab, rtab_v)

    copies = []
    for j in range(NCHUNK):
        sl = pl.ds(j * IC, IC)
        copies.append((
            pltpu.async_copy(ent_tab.at[eidx_v.at[sl]], erows_v.at[sl], sems[j]),
            pltpu.async_copy(ent_tab.at[vidx_v.at[sl]], vrows_v.at[sl], sems[j]),
        ))

    z = jnp.zeros((LANES,), jnp.float32)

    def zbody(r, _):
        for c in range(CH):
            acc_v[r, pl.ds(c * LANES, LANES)] = z
        return 0

    lax.fori_loop(0, N_REL, zbody, 0)

    copies[0][0].wait()
    copies[0][1].wait()

    def gbody(g, _):
        for j in range(1, NCHUNK):
            @pl.when(g == j * GPC)
            def _wait(j=j):
                copies[j][0].wait()
                copies[j][1].wait()

        rv = ridx_v[pl.ds(g * LANES, LANES)]
        for l in range(LANES):
            r = rv[l]
            i = g * LANES + l
            for c in range(D // 32):
                ev = erows_v[i, pl.ds(c * 32, 32)]
                vv = vrows_v[i, pl.ds(c * 32, 32)]
                ea, eb = plsc.unpack(ev, format=plsc.PackFormat.INTERLEAVED,
                                     preferred_element_type=jnp.float32)
                va, vb = plsc.unpack(vv, format=plsc.PackFormat.INTERLEAVED,
                                     preferred_element_type=jnp.float32)
                plsc.addupdate(acc_v.at[r, pl.ds(c * 32, LANES)], ea + va)
                plsc.addupdate(acc_v.at[r, pl.ds(c * 32 + LANES, LANES)], eb + vb)
        return 0

    lax.fori_loop(0, NG, gbody, 0)

    def fbody(r, part):
        out_part = []
        for c in range(CH):
            sl = pl.ds(c * LANES, LANES)
            out_part.append(part[c] + rtab_v[r, sl] * acc_v[r, sl])
        return tuple(out_part)

    part = lax.fori_loop(0, N_REL, fbody, (z,) * CH)
    for c in range(CH):
        part_v[pl.ds(c * LANES, LANES)] = part[c]
    pltpu.sync_copy(part_v, out.at[wid])


@jax.jit
def kernel(rel_idx, ent_idx, val_idx, relation_embed, entity_embed):
    mesh = plsc.VectorSubcoreMesh(core_axis_name="c", subcore_axis_name="s")
    k = functools.partial(
        pl.kernel,
        mesh=mesh,
        compiler_params=pltpu.CompilerParams(
            use_tc_tiling_on_sc=False, needs_layout_passes=False),
        out_type=jax.ShapeDtypeStruct((NW, D), jnp.float32),
        scratch_types=[
            pltpu.VMEM((TPW,), jnp.int32),
            pltpu.VMEM((TPW,), jnp.int32),
            pltpu.VMEM((TPW,), jnp.int32),
            pltpu.VMEM((N_REL, D), jnp.float32),
            pltpu.VMEM((TPW, D), jnp.bfloat16),
            pltpu.VMEM((TPW, D), jnp.bfloat16),
            pltpu.VMEM((N_REL, D), jnp.float32),
            pltpu.VMEM((D,), jnp.float32),
            pltpu.SemaphoreType.DMA,
            pltpu.SemaphoreType.DMA,
            pltpu.SemaphoreType.DMA,
            pltpu.SemaphoreType.DMA,
        ],
    )(_body)
    partials = k(
        rel_idx,
        ent_idx,
        val_idx,
        relation_embed[:, _PERM],
        entity_embed.astype(jnp.bfloat16),
    )
    return partials.sum(axis=0)[_INV_PERM] * (1.0 / N)


# speedup vs baseline: 1.0916x; 1.0175x over previous
import functools

import jax
import jax.numpy as jnp
from jax import lax
from jax.experimental import pallas as pl
from jax.experimental.pallas import tpu as pltpu
from jax.experimental.pallas import tpu_sc as plsc

N = 16384
D = 64
N_REL = 10
NC = 2
NS = 16
NW = NC * NS
TPW = N // NW
IC = 128
NCHUNK = TPW // IC
LANES = 16
CH = D // LANES
GPC = IC // LANES
NG = TPW // LANES

def _body(idx, rel_tab, ent_tab, out,
          ridx_v, eidx_v, vidx_v, rtab_v, erows_v, vrows_v, acc_v, part_v,
          sem0, sem1, sem2, sem3):
    sems = (sem0, sem1, sem2, sem3)
    wid = lax.axis_index("s") * NC + lax.axis_index("c")
    base = wid * TPW

    pltpu.sync_copy(idx.at[pl.ds(base, TPW)], ridx_v)
    pltpu.sync_copy(idx.at[pl.ds(N + base, TPW)], eidx_v)
    pltpu.sync_copy(idx.at[pl.ds(2 * N + base, TPW)], vidx_v)
    pltpu.sync_copy(rel_tab, rtab_v)

    copies = []
    for j in range(NCHUNK):
        sl = pl.ds(j * IC, IC)
        copies.append((
            pltpu.async_copy(ent_tab.at[eidx_v.at[sl]], erows_v.at[sl], sems[j]),
            pltpu.async_copy(ent_tab.at[vidx_v.at[sl]], vrows_v.at[sl], sems[j]),
        ))

    z = jnp.zeros((LANES,), jnp.float32)

    def zbody(r, _):
        for c in range(CH):
            acc_v[r, pl.ds(c * LANES, LANES)] = z
        return 0

    lax.fori_loop(0, N_REL, zbody, 0)

    copies[0][0].wait()
    copies[0][1].wait()

    def gbody(g, _):
        for j in range(1, NCHUNK):
            @pl.when(g == j * GPC)
            def _wait(j=j):
                copies[j][0].wait()
                copies[j][1].wait()

        rv = ridx_v[pl.ds(g * LANES, LANES)]
        for l in range(LANES):
            r = rv[l]
            i = g * LANES + l
            for c in range(D // 32):
                ev = erows_v[i, pl.ds(c * 32, 32)]
                vv = vrows_v[i, pl.ds(c * 32, 32)]
                ea, eb = plsc.unpack(ev, format=plsc.PackFormat.INTERLEAVED,
                                     preferred_element_type=jnp.float32)
                va, vb = plsc.unpack(vv, format=plsc.PackFormat.INTERLEAVED,
                                     preferred_element_type=jnp.float32)
                plsc.addupdate(acc_v.at[r, pl.ds(c * 32, LANES)], ea + va)
                plsc.addupdate(acc_v.at[r, pl.ds(c * 32 + LANES, LANES)], eb + vb)
        return 0

    lax.fori_loop(0, NG, gbody, 0)

    iota = lax.iota(jnp.int32, LANES)

    def fbody(r, part):
        rows = jnp.full((LANES,), r, jnp.int32)
        out_part = []
        for c in range(CH):
            cols = (c // 2) * 32 + 2 * iota + (c % 2)
            rrow = plsc.load_gather(rtab_v, [rows, cols])
            out_part.append(part[c] + rrow * acc_v[r, pl.ds(c * LANES, LANES)])
        return tuple(out_part)

    part = lax.fori_loop(0, N_REL, fbody, (z,) * CH)
    for c in range(CH):
        cols = (c // 2) * 32 + 2 * iota + (c % 2)
        plsc.store_scatter(part_v, [cols], part[c])
    pltpu.sync_copy(part_v, out.at[wid])


@jax.jit
def kernel(rel_idx, ent_idx, val_idx, relation_embed, entity_embed):
    mesh = plsc.VectorSubcoreMesh(core_axis_name="c", subcore_axis_name="s")
    k = functools.partial(
        pl.kernel,
        mesh=mesh,
        compiler_params=pltpu.CompilerParams(
            use_tc_tiling_on_sc=False, needs_layout_passes=False),
        out_type=jax.ShapeDtypeStruct((NW, D), jnp.float32),
        scratch_types=[
            pltpu.VMEM((TPW,), jnp.int32),
            pltpu.VMEM((TPW,), jnp.int32),
            pltpu.VMEM((TPW,), jnp.int32),
            pltpu.VMEM((N_REL, D), jnp.float32),
            pltpu.VMEM((TPW, D), jnp.bfloat16),
            pltpu.VMEM((TPW, D), jnp.bfloat16),
            pltpu.VMEM((N_REL, D), jnp.float32),
            pltpu.VMEM((D,), jnp.float32),
            pltpu.SemaphoreType.DMA,
            pltpu.SemaphoreType.DMA,
            pltpu.SemaphoreType.DMA,
            pltpu.SemaphoreType.DMA,
        ],
    )(_body)
    partials = k(
        jnp.concatenate([rel_idx, ent_idx, val_idx]),
        relation_embed,
        entity_embed.astype(jnp.bfloat16),
    )
    return partials.sum(axis=0) * (1.0 / N)
